# baseline (speedup 1.0000x reference)
import functools
import math

import jax
import jax.numpy as jnp
from jax import lax
from jax.experimental import pallas as pl
from jax.experimental.pallas import tpu as pltpu
from jax.experimental.pallas import tpu_sc as plsc

N_NODES = 10000
N_EDGES = 320000
G_GRAPHS = 64

TILES = 16
CHUNK = 125
RING = 3
IDXR = 6
STRIPE = 640


def _pieces(total):
    out, off = [], 0
    while off < total:
        n = min(120, total - off)
        out.append((off, n))
        off += n
    return out


def _pad_chunks(n):
    return -(-n // IDXR) * IDXR


def _build_sc_agg(split_cols, interpret=False):
    width = 128
    n_slices = TILES if split_cols else 2 * TILES
    n_chunks = _pad_chunks(N_EDGES // n_slices // CHUNK)
    mesh = plsc.VectorSubcoreMesh(core_axis_name="c", subcore_axis_name="s",
                                  num_cores=2, num_subcores=TILES)

    def body(hL_hbm, hR_hbm, se_hbm, outL_hbm, outR_hbm,
             idx, r0, r1, r2, shared_agg,
             sg0, sg1, sg2, ss0, ss1, ss2, semi):
        rows = (r0, r1, r2)
        semg = (sg0, sg1, sg2)
        sems = (ss0, ss1, ss2)
        cid = lax.axis_index("c")
        sid = lax.axis_index("s")
        n16 = width // 16

        def zero_r0(k, _):
            i = k // n16
            j = k - i * n16
            r0[i, pl.ds(j * 16, 16)] = jnp.zeros((16,), jnp.float32)
            return 0
        lax.fori_loop(0, CHUNK * n16, zero_r0, 0)

        def stripe_copy(sub_fn):
            base = sid * STRIPE

            @pl.when(sid < TILES - 1)
            def _():
                for off, n in _pieces(STRIPE):
                    sub_fn(base + off, n)

            @pl.when(sid == TILES - 1)
            def _():
                for off, n in _pieces(N_NODES - (TILES - 1) * STRIPE):
                    sub_fn(base + off, n)

        for ci in range(2):
            h_ref = (hL_hbm, hR_hbm)[ci] if split_cols else hL_hbm
            out_ref = (outL_hbm, outR_hbm)[ci]
            tslice = sid if split_cols else ci * TILES + sid

            @pl.when(cid == ci)
            def _():
                def zero_sub(off, n):
                    pltpu.sync_copy(r0.at[pl.ds(0, n)],
                                    shared_agg.at[pl.ds(off, n)])
                stripe_copy(zero_sub)
                plsc.subcore_barrier()

                def idx_copy(k, s):
                    return pltpu.make_async_copy(se_hbm.at[tslice, k],
                                                 idx.at[s], semi)

                def gather(s, b):
                    return pltpu.make_async_copy(h_ref.at[idx.at[s, 0]],
                                                 rows[b], semg[b])

                def scat_wait(s, b):
                    pltpu.make_async_copy(
                        rows[b], shared_agg.at[idx.at[s, 1]], sems[b]).wait()

                def scat_start(s, b):
                    pltpu.async_copy(
                        rows[b], shared_agg.at[idx.at[s, 1]], sems[b],
                        add=True)

                idx_copy(0, 0).start()
                idx_copy(0, 0).wait()
                gather(0, 0).start()
                idx_copy(1, 1).start()

                def step(k, j):
                    b = j % RING

                    @pl.when(k + 1 < n_chunks)
                    def _():
                        idx_copy(k + 1, (j + 1) % IDXR).wait()

                    @pl.when(k >= 2)
                    def _():
                        scat_wait((j - 2) % IDXR, (j - 2) % RING)

                    @pl.when(k + 1 < n_chunks)
                    def _():
                        gather((j + 1) % IDXR, (j + 1) % RING).start()

                    @pl.when(k + 2 < n_chunks)
                    def _():
                        idx_copy(k + 2, (j + 2) % IDXR).start()
                    gather(j % IDXR, b).wait()
                    scat_start(j % IDXR, b)

                def sweep(p, _):
                    for j in range(IDXR):
                        step(IDXR * p + j, j)
                    return 0
                lax.fori_loop(0, n_chunks // IDXR, sweep, 0)
                scat_wait((n_chunks - 2) % IDXR, (n_chunks - 2) % RING)
                scat_wait((n_chunks - 1) % IDXR, (n_chunks - 1) % RING)

                plsc.subcore_barrier()

                def out_sub(off, n):
                    pltpu.sync_copy(shared_agg.at[pl.ds(off, n)],
                                    out_ref.at[pl.ds(off, n)])
                stripe_copy(out_sub)

    def wrapped(hL, hR, se):
        return pl.kernel(
            body,
            out_type=[jax.ShapeDtypeStruct((N_NODES, width), jnp.float32),
                      jax.ShapeDtypeStruct((N_NODES, width), jnp.float32)],
            mesh=mesh,
            scratch_types=[
                pltpu.VMEM((IDXR, 2, CHUNK), jnp.int32),
                pltpu.VMEM((CHUNK, width), jnp.float32),
                pltpu.VMEM((CHUNK, width), jnp.float32),
                pltpu.VMEM((CHUNK, width), jnp.float32),
                pltpu.VMEM_SHARED((N_NODES + 8, width), jnp.float32),
                pltpu.SemaphoreType.DMA,
                pltpu.SemaphoreType.DMA,
                pltpu.SemaphoreType.DMA,
                pltpu.SemaphoreType.DMA,
                pltpu.SemaphoreType.DMA,
                pltpu.SemaphoreType.DMA,
                pltpu.SemaphoreType.DMA,
            ],
            interpret=interpret,
        )(hL, hR, se)

    return wrapped


def _stack_edges(edge_index, n_slices):
    n_real = N_EDGES // n_slices // CHUNK
    n_pad = _pad_chunks(n_real) - n_real
    se = jnp.stack([edge_index[0].reshape(n_slices, n_real, CHUNK),
                    edge_index[1].reshape(n_slices, n_real, CHUNK)], axis=2)
    if n_pad:
        pad = jnp.zeros((n_slices, n_pad, 2, CHUNK), jnp.int32)
        pad = pad.at[:, :, 1, :].set(N_NODES)
        se = jnp.concatenate([se, pad], axis=1)
    return se


BR = 1000
N_BLOCKS = N_NODES // BR
BN_SCALE = 1.0 / math.sqrt(1.0 + 1e-5)


def _mlp_tail(z, W2, b2, gm, bt, outL, outR):
    a = jnp.maximum(z, 0.0)
    z2 = jnp.dot(a, W2[...], preferred_element_type=jnp.float32) + b2[...]
    h = z2 * (BN_SCALE * gm[...]) + bt[...]
    h = jnp.maximum(h, 0.0)
    outL[...] = h[:, :128]
    outR[...] = h[:, 128:]


def _mlp_body(hL, hR, aL, aR, W1, b1, W2, b2, gm, bt, outL, outR):
    w1 = W1[...]
    z = (jnp.dot(hL[...] + aL[...], w1[:128, :],
                 preferred_element_type=jnp.float32)
         + jnp.dot(hR[...] + aR[...], w1[128:, :],
                   preferred_element_type=jnp.float32)
         + b1[...])
    _mlp_tail(z, W2, b2, gm, bt, outL, outR)


def _mlp0_body(x, aA, aB, W1, b1, W2, b2, gm, bt, outL, outR):
    u = x[...] + aA[...] + aB[...]
    z = jnp.dot(u, W1[...], preferred_element_type=jnp.float32) + b1[...]
    _mlp_tail(z, W2, b2, gm, bt, outL, outR)


def _build_mlp(first, interpret=False):
    din = 128 if first else 256
    return pl.pallas_call(
        _mlp0_body if first else _mlp_body,
        grid=(N_BLOCKS,),
        in_specs=[
            pl.BlockSpec((BR, 128), lambda i: (i, 0)),
            pl.BlockSpec((BR, 128), lambda i: (i, 0)),
            pl.BlockSpec((BR, 128), lambda i: (i, 0)),
            pl.BlockSpec((BR, 128), lambda i: (i, 0)),
            pl.BlockSpec((din, 256), lambda i: (0, 0)),
            pl.BlockSpec((1, 256), lambda i: (0, 0)),
            pl.BlockSpec((256, 256), lambda i: (0, 0)),
            pl.BlockSpec((1, 256), lambda i: (0, 0)),
            pl.BlockSpec((1, 256), lambda i: (0, 0)),
            pl.BlockSpec((1, 256), lambda i: (0, 0)),
        ][0 if not first else 1:],
        out_specs=[
            pl.BlockSpec((BR, 128), lambda i: (i, 0)),
            pl.BlockSpec((BR, 128), lambda i: (i, 0)),
        ],
        out_shape=[jax.ShapeDtypeStruct((N_NODES, 128), jnp.float32),
                   jax.ShapeDtypeStruct((N_NODES, 128), jnp.float32)],
        interpret=interpret,
    )


def _final_body(h0L, h0R, h1L, h1R, h2L, h2R, batch, Wj, bj, Wc1, bc1,
                Wc2, bc2, out, acc, cnt):
    i = pl.program_id(0)

    @pl.when(i == 0)
    def _():
        acc[...] = jnp.zeros_like(acc)
        cnt[...] = jnp.zeros_like(cnt)

    hcat = jnp.concatenate(
        [h0L[...], h0R[...], h1L[...], h1R[...], h2L[...], h2R[...]], axis=1)
    seg = jax.lax.broadcasted_iota(jnp.int32, (1, G_GRAPHS), 1)
    onehot = (batch[...] == seg).astype(jnp.float32)
    acc[...] += lax.dot_general(onehot, hcat,
                                (((0,), (0,)), ((), ())),
                                preferred_element_type=jnp.float32)
    ones = jnp.ones((BR, 1), jnp.float32)
    cnt[...] += lax.dot_general(onehot, ones,
                                (((0,), (0,)), ((), ())),
                                preferred_element_type=jnp.float32)

    @pl.when(i == N_BLOCKS - 1)
    def _():
        pooled = (jnp.dot(acc[...], Wj[...],
                          preferred_element_type=jnp.float32)
                  + cnt[...] * bj[...])
        q = jnp.maximum(
            jnp.dot(pooled, Wc1[...], preferred_element_type=jnp.float32)
            + bc1[...], 0.0)
        out[...] = (jnp.dot(q, Wc2[...], preferred_element_type=jnp.float32)
                    + bc2[...])


def _build_final(interpret=False):
    hspec = pl.BlockSpec((BR, 128), lambda i: (i, 0))
    return pl.pallas_call(
        _final_body,
        grid=(N_BLOCKS,),
        in_specs=[
            hspec, hspec, hspec, hspec, hspec, hspec,
            pl.BlockSpec((BR, 1), lambda i: (i, 0)),
            pl.BlockSpec((768, 256), lambda i: (0, 0)),
            pl.BlockSpec((1, 256), lambda i: (0, 0)),
            pl.BlockSpec((256, 128), lambda i: (0, 0)),
            pl.BlockSpec((1, 128), lambda i: (0, 0)),
            pl.BlockSpec((128, 128), lambda i: (0, 0)),
            pl.BlockSpec((1, 128), lambda i: (0, 0)),
        ],
        out_specs=pl.BlockSpec((G_GRAPHS, 128), lambda i: (0, 0)),
        out_shape=jax.ShapeDtypeStruct((G_GRAPHS, 128), jnp.float32),
        scratch_shapes=[
            pltpu.VMEM((G_GRAPHS, 768), jnp.float32),
            pltpu.VMEM((G_GRAPHS, 1), jnp.float32),
        ],
        interpret=interpret,
    )


def _run(x, edge_index, batch, params, jump, head, *, interpret=False):
    se_s = _stack_edges(edge_index, TILES)
    se_f = _stack_edges(edge_index, 2 * TILES)
    batch2 = batch.reshape(N_NODES, 1)

    sc_full = _build_sc_agg(False, interpret=interpret)
    sc_split = _build_sc_agg(True, interpret=interpret)
    mlp0 = _build_mlp(True, interpret=interpret)
    mlp = _build_mlp(False, interpret=interpret)
    fin = _build_final(interpret=interpret)

    def row(v):
        return v.reshape(1, -1)

    halves = []
    hL = hR = None
    for li, (W1, b1, W2, b2, gm, bt) in enumerate(params):
        if li == 0:
            aA, aB = sc_full(x, x, se_f)
            hL, hR = mlp0(x, aA, aB, W1, row(b1), W2, row(b2),
                          row(gm), row(bt))
        else:
            aL, aR = sc_split(hL, hR, se_s)
            hL, hR = mlp(hL, hR, aL, aR, W1, row(b1), W2, row(b2),
                         row(gm), row(bt))
        halves.extend([hL, hR])

    Wj, bj = jump
    Wc1, bc1, Wc2, bc2 = head
    Wc2p = jnp.pad(Wc2, ((0, 0), (0, 128 - Wc2.shape[1])))
    bc2p = jnp.pad(bc2, (0, 128 - bc2.shape[0]))
    outp = fin(*halves, batch2, Wj, row(bj), Wc1, row(bc1), Wc2p, row(bc2p))
    return outp[:, :Wc2.shape[1]]


def kernel(x, edge_index, batch, W1_0, b1_0, W2_0, b2_0, gamma_0, beta_0,
           W1_1, b1_1, W2_1, b2_1, gamma_1, beta_1,
           W1_2, b1_2, W2_2, b2_2, gamma_2, beta_2,
           Wj, bj, Wc1, bc1, Wc2, bc2):
    params = [
        (W1_0, b1_0, W2_0, b2_0, gamma_0, beta_0),
        (W1_1, b1_1, W2_1, b2_1, gamma_1, beta_1),
        (W1_2, b1_2, W2_2, b2_2, gamma_2, beta_2),
    ]
    return _run(x, edge_index, batch, params, (Wj, bj), (Wc1, bc1, Wc2, bc2))


# speedup vs baseline: 2.1277x; 2.1277x over previous
import functools
import math

import jax
import jax.numpy as jnp
from jax import lax
from jax.experimental import pallas as pl
from jax.experimental.pallas import tpu as pltpu
from jax.experimental.pallas import tpu_sc as plsc

N_NODES = 10000
N_EDGES = 320000
G_GRAPHS = 64

TILES = 16
CHUNK = 125
STRIPE = 640


def _pieces(total):
    out, off = [], 0
    while off < total:
        n = min(120, total - off)
        out.append((off, n))
        off += n
    return out


def _pad_chunks(n):
    return -(-n // 2) * 2


def _build_sc_agg(split_cols, interpret=False):
    width = 128
    n_slices = TILES if split_cols else 2 * TILES
    n_chunks = _pad_chunks(N_EDGES // n_slices // CHUNK)
    mesh = plsc.VectorSubcoreMesh(core_axis_name="c", subcore_axis_name="s",
                                  num_cores=2, num_subcores=TILES)

    def body(hL_hbm, hR_hbm, se_hbm, outL_hbm, outR_hbm,
             idx, r0, r1, shared_agg, sg0, sg1, semi):
        rows = (r0, r1)
        semg = (sg0, sg1)
        cid = lax.axis_index("c")
        sid = lax.axis_index("s")
        n16 = width // 16

        def zero_r0(k, _):
            i = k // n16
            j = k - i * n16
            r0[i, pl.ds(j * 16, 16)] = jnp.zeros((16,), jnp.float32)
            return 0
        lax.fori_loop(0, CHUNK * n16, zero_r0, 0)

        def stripe_copy(sub_fn):
            base = sid * STRIPE

            @pl.when(sid < TILES - 1)
            def _():
                for off, n in _pieces(STRIPE):
                    sub_fn(base + off, n)

            @pl.when(sid == TILES - 1)
            def _():
                for off, n in _pieces(N_NODES - (TILES - 1) * STRIPE):
                    sub_fn(base + off, n)

        for ci in range(2):
            h_ref = (hL_hbm, hR_hbm)[ci] if split_cols else hL_hbm
            out_ref = (outL_hbm, outR_hbm)[ci]
            tslice = sid if split_cols else ci * TILES + sid

            @pl.when(cid == ci)
            def _():
                def zero_sub(off, n):
                    pltpu.sync_copy(r0.at[pl.ds(0, n)],
                                    shared_agg.at[pl.ds(off, n)])
                stripe_copy(zero_sub)
                plsc.subcore_barrier()

                def idx_copy(k, s):
                    return pltpu.make_async_copy(se_hbm.at[tslice, k],
                                                 idx.at[s], semi)

                def gather(b):
                    return pltpu.make_async_copy(h_ref.at[idx.at[b, 0]],
                                                 rows[b], semg[b])

                idx_copy(0, 0).start()
                idx_copy(0, 0).wait()
                gather(0).start()
                idx_copy(1, 1).start()

                def step(k, q):
                    @pl.when(k + 1 < n_chunks)
                    def _():
                        idx_copy(k + 1, 1 - q).wait()
                        gather(1 - q).start()
                    gather(q).wait()
                    pltpu.sync_copy(rows[q], shared_agg.at[idx.at[q, 1]],
                                    add=True)

                    @pl.when(k + 2 < n_chunks)
                    def _():
                        idx_copy(k + 2, q).start()

                def pair(p, _):
                    step(2 * p, 0)
                    step(2 * p + 1, 1)
                    return 0
                lax.fori_loop(0, n_chunks // 2, pair, 0)

                plsc.subcore_barrier()

                def out_sub(off, n):
                    pltpu.sync_copy(shared_agg.at[pl.ds(off, n)],
                                    out_ref.at[pl.ds(off, n)])
                stripe_copy(out_sub)

    def wrapped(hL, hR, se):
        return pl.kernel(
            body,
            out_type=[jax.ShapeDtypeStruct((N_NODES, width), jnp.float32),
                      jax.ShapeDtypeStruct((N_NODES, width), jnp.float32)],
            mesh=mesh,
            scratch_types=[
                pltpu.VMEM((2, 2, CHUNK), jnp.int32),
                pltpu.VMEM((CHUNK, width), jnp.float32),
                pltpu.VMEM((CHUNK, width), jnp.float32),
                pltpu.VMEM_SHARED((N_NODES + 8, width), jnp.float32),
                pltpu.SemaphoreType.DMA,
                pltpu.SemaphoreType.DMA,
                pltpu.SemaphoreType.DMA,
            ],
            interpret=interpret,
        )(hL, hR, se)

    return wrapped


def _stack_edges(edge_index, n_slices):
    n_real = N_EDGES // n_slices // CHUNK
    n_pad = _pad_chunks(n_real) - n_real
    se = jnp.stack([edge_index[0].reshape(n_slices, n_real, CHUNK),
                    edge_index[1].reshape(n_slices, n_real, CHUNK)], axis=2)
    if n_pad:
        pad = jnp.zeros((n_slices, n_pad, 2, CHUNK), jnp.int32)
        pad = pad.at[:, :, 1, :].set(N_NODES)
        se = jnp.concatenate([se, pad], axis=1)
    return se


BR = 1000
N_BLOCKS = N_NODES // BR
BN_SCALE = 1.0 / math.sqrt(1.0 + 1e-5)


def _mlp_tail(z, W2, b2, gm, bt, outL, outR):
    a = jnp.maximum(z, 0.0)
    z2 = jnp.dot(a, W2[...], preferred_element_type=jnp.float32) + b2[...]
    h = z2 * (BN_SCALE * gm[...]) + bt[...]
    h = jnp.maximum(h, 0.0)
    outL[...] = h[:, :128]
    outR[...] = h[:, 128:]


def _mlp_body(hL, hR, aL, aR, W1, b1, W2, b2, gm, bt, outL, outR):
    w1 = W1[...]
    z = (jnp.dot(hL[...] + aL[...], w1[:128, :],
                 preferred_element_type=jnp.float32)
         + jnp.dot(hR[...] + aR[...], w1[128:, :],
                   preferred_element_type=jnp.float32)
         + b1[...])
    _mlp_tail(z, W2, b2, gm, bt, outL, outR)


def _mlp0_body(x, aA, aB, W1, b1, W2, b2, gm, bt, outL, outR):
    u = x[...] + aA[...] + aB[...]
    z = jnp.dot(u, W1[...], preferred_element_type=jnp.float32) + b1[...]
    _mlp_tail(z, W2, b2, gm, bt, outL, outR)


def _build_mlp(first, interpret=False):
    din = 128 if first else 256
    return pl.pallas_call(
        _mlp0_body if first else _mlp_body,
        grid=(N_BLOCKS,),
        in_specs=[
            pl.BlockSpec((BR, 128), lambda i: (i, 0)),
            pl.BlockSpec((BR, 128), lambda i: (i, 0)),
            pl.BlockSpec((BR, 128), lambda i: (i, 0)),
            pl.BlockSpec((BR, 128), lambda i: (i, 0)),
            pl.BlockSpec((din, 256), lambda i: (0, 0)),
            pl.BlockSpec((1, 256), lambda i: (0, 0)),
            pl.BlockSpec((256, 256), lambda i: (0, 0)),
            pl.BlockSpec((1, 256), lambda i: (0, 0)),
            pl.BlockSpec((1, 256), lambda i: (0, 0)),
            pl.BlockSpec((1, 256), lambda i: (0, 0)),
        ][0 if not first else 1:],
        out_specs=[
            pl.BlockSpec((BR, 128), lambda i: (i, 0)),
            pl.BlockSpec((BR, 128), lambda i: (i, 0)),
        ],
        out_shape=[jax.ShapeDtypeStruct((N_NODES, 128), jnp.float32),
                   jax.ShapeDtypeStruct((N_NODES, 128), jnp.float32)],
        interpret=interpret,
    )


def _final_body(h0L, h0R, h1L, h1R, h2L, h2R, batch, Wj, bj, Wc1, bc1,
                Wc2, bc2, out, acc, cnt):
    i = pl.program_id(0)

    @pl.when(i == 0)
    def _():
        acc[...] = jnp.zeros_like(acc)
        cnt[...] = jnp.zeros_like(cnt)

    hcat = jnp.concatenate(
        [h0L[...], h0R[...], h1L[...], h1R[...], h2L[...], h2R[...]], axis=1)
    seg = jax.lax.broadcasted_iota(jnp.int32, (1, G_GRAPHS), 1)
    onehot = (batch[...] == seg).astype(jnp.float32)
    acc[...] += lax.dot_general(onehot, hcat,
                                (((0,), (0,)), ((), ())),
                                preferred_element_type=jnp.float32)
    ones = jnp.ones((BR, 1), jnp.float32)
    cnt[...] += lax.dot_general(onehot, ones,
                                (((0,), (0,)), ((), ())),
                                preferred_element_type=jnp.float32)

    @pl.when(i == N_BLOCKS - 1)
    def _():
        pooled = (jnp.dot(acc[...], Wj[...],
                          preferred_element_type=jnp.float32)
                  + cnt[...] * bj[...])
        q = jnp.maximum(
            jnp.dot(pooled, Wc1[...], preferred_element_type=jnp.float32)
            + bc1[...], 0.0)
        out[...] = (jnp.dot(q, Wc2[...], preferred_element_type=jnp.float32)
                    + bc2[...])


def _build_final(interpret=False):
    hspec = pl.BlockSpec((BR, 128), lambda i: (i, 0))
    return pl.pallas_call(
        _final_body,
        grid=(N_BLOCKS,),
        in_specs=[
            hspec, hspec, hspec, hspec, hspec, hspec,
            pl.BlockSpec((BR, 1), lambda i: (i, 0)),
            pl.BlockSpec((768, 256), lambda i: (0, 0)),
            pl.BlockSpec((1, 256), lambda i: (0, 0)),
            pl.BlockSpec((256, 128), lambda i: (0, 0)),
            pl.BlockSpec((1, 128), lambda i: (0, 0)),
            pl.BlockSpec((128, 128), lambda i: (0, 0)),
            pl.BlockSpec((1, 128), lambda i: (0, 0)),
        ],
        out_specs=pl.BlockSpec((G_GRAPHS, 128), lambda i: (0, 0)),
        out_shape=jax.ShapeDtypeStruct((G_GRAPHS, 128), jnp.float32),
        scratch_shapes=[
            pltpu.VMEM((G_GRAPHS, 768), jnp.float32),
            pltpu.VMEM((G_GRAPHS, 1), jnp.float32),
        ],
        interpret=interpret,
    )


def _run(x, edge_index, batch, params, jump, head, *, interpret=False):
    se_s = _stack_edges(edge_index, TILES)
    se_f = _stack_edges(edge_index, 2 * TILES)
    batch2 = batch.reshape(N_NODES, 1)

    sc_full = _build_sc_agg(False, interpret=interpret)
    sc_split = _build_sc_agg(True, interpret=interpret)
    mlp0 = _build_mlp(True, interpret=interpret)
    mlp = _build_mlp(False, interpret=interpret)
    fin = _build_final(interpret=interpret)

    def row(v):
        return v.reshape(1, -1)

    halves = []
    hL = hR = None
    for li, (W1, b1, W2, b2, gm, bt) in enumerate(params):
        if li == 0:
            aA, aB = sc_full(x, x, se_f)
            hL, hR = mlp0(x, aA, aB, W1, row(b1), W2, row(b2),
                          row(gm), row(bt))
        else:
            aL, aR = sc_split(hL, hR, se_s)
            hL, hR = mlp(hL, hR, aL, aR, W1, row(b1), W2, row(b2),
                         row(gm), row(bt))
        halves.extend([hL, hR])

    Wj, bj = jump
    Wc1, bc1, Wc2, bc2 = head
    Wc2p = jnp.pad(Wc2, ((0, 0), (0, 128 - Wc2.shape[1])))
    bc2p = jnp.pad(bc2, (0, 128 - bc2.shape[0]))
    outp = fin(*halves, batch2, Wj, row(bj), Wc1, row(bc1), Wc2p, row(bc2p))
    return outp[:, :Wc2.shape[1]]


def kernel(x, edge_index, batch, W1_0, b1_0, W2_0, b2_0, gamma_0, beta_0,
           W1_1, b1_1, W2_1, b2_1, gamma_1, beta_1,
           W1_2, b1_2, W2_2, b2_2, gamma_2, beta_2,
           Wj, bj, Wc1, bc1, Wc2, bc2):
    params = [
        (W1_0, b1_0, W2_0, b2_0, gamma_0, beta_0),
        (W1_1, b1_1, W2_1, b2_1, gamma_1, beta_1),
        (W1_2, b1_2, W2_2, b2_2, gamma_2, beta_2),
    ]
    return _run(x, edge_index, batch, params, (Wj, bj), (Wc1, bc1, Wc2, bc2))
